# baseline (speedup 1.0000x reference)
import jax
import jax.numpy as jnp
from jax import lax
from jax.experimental import pallas as pl
from jax.experimental.pallas import tpu as pltpu

D_MODEL = 1024
NUM_EXPERTS = 8
SEQ = 2048
EPAIR = 1
NPAIR = NUM_EXPERTS // EPAIR


def _moe_body(x_ref, shared_W_ref, shared_b_ref, router_W_ref,
              router_b_ref, expert_W_ref, eb_all_ref, out_ref, coeff_ref):
    u = pl.program_id(0)

    @pl.when(u == 0)
    def _():
        xb = x_ref[...]
        scores = lax.dot_general(xb, router_W_ref[...],
                                 (((1,), (1,)), ((), ())),
                                 preferred_element_type=jnp.float32)
        scores = scores + router_b_ref[...]
        eidx = lax.broadcasted_iota(jnp.int32, scores.shape, 1)
        m0 = jnp.max(scores, axis=-1, keepdims=True)
        a0 = jnp.min(jnp.where(scores == m0, eidx, NUM_EXPERTS), axis=-1,
                     keepdims=True)
        masked = jnp.where(eidx == a0, -jnp.inf, scores)
        m1 = jnp.max(masked, axis=-1, keepdims=True)
        a1 = jnp.min(jnp.where(masked == m1, eidx, NUM_EXPERTS), axis=-1,
                     keepdims=True)
        z = jnp.exp(m1 - m0)
        w0 = 1.0 / (1.0 + z)
        w1 = z * w0
        coeff = (jnp.where(eidx == a0, w0, 0.0)
                 + jnp.where(eidx == a1, w1, 0.0))
        coeff_ref[...] = coeff
        so = lax.dot_general(xb, shared_W_ref[...], (((1,), (1,)), ((), ())),
                             preferred_element_type=jnp.float32)
        bias_mix = lax.dot_general(coeff, eb_all_ref[...],
                                   (((1,), (0,)), ((), ())),
                                   preferred_element_type=jnp.float32)
        out_ref[...] = so + shared_b_ref[...] + bias_mix

    @pl.when(u > 0)
    def _():
        xb = x_ref[...]
        call = coeff_ref[...]
        lane = lax.broadcasted_iota(jnp.int32, call.shape, 1)
        for k in range(EPAIR):
            e = (u - 1) * EPAIR + k
            eo = lax.dot_general(xb, expert_W_ref[k],
                                 (((1,), (1,)), ((), ())),
                                 preferred_element_type=jnp.float32)
            coeff = jnp.sum(jnp.where(lane == e, call, 0.0), axis=1,
                            keepdims=True)
            out_ref[...] += coeff * eo


@jax.jit
def kernel(x, shared_W, shared_b, router_W, router_b, expert_W, expert_b):
    B, S, D = x.shape
    x2 = x.reshape(S, D)

    def _w_idx(u):
        e = jnp.maximum(u - 1, 0)
        return (e, 0, 0)

    out = pl.pallas_call(
        _moe_body,
        grid=(NPAIR + 1,),
        in_specs=[
            pl.BlockSpec((S, D), lambda u: (0, 0)),
            pl.BlockSpec((D, D), lambda u: (0, 0)),
            pl.BlockSpec((1, D), lambda u: (0, 0)),
            pl.BlockSpec((NUM_EXPERTS, D), lambda u: (0, 0)),
            pl.BlockSpec((1, NUM_EXPERTS), lambda u: (0, 0)),
            pl.BlockSpec((EPAIR, D, D), _w_idx),
            pl.BlockSpec((NUM_EXPERTS, D), lambda u: (0, 0)),
        ],
        out_specs=pl.BlockSpec((S, D), lambda u: (0, 0)),
        out_shape=jax.ShapeDtypeStruct((S, D), jnp.float32),
        scratch_shapes=[pltpu.VMEM((S, NUM_EXPERTS), jnp.float32)],
    )(x2, shared_W, shared_b.reshape(1, D),
      router_W, router_b.reshape(1, NUM_EXPERTS),
      expert_W, expert_b)
    return out.reshape(B, S, D)


# speedup vs baseline: 1.0387x; 1.0387x over previous
import jax
import jax.numpy as jnp
from jax import lax
from jax.experimental import pallas as pl
from jax.experimental.pallas import tpu as pltpu

D_MODEL = 1024
NUM_EXPERTS = 8
SEQ = 2048
EPAIR = 1
NPAIR = NUM_EXPERTS // EPAIR


def _moe_body(x_ref, shared_W_ref, shared_b_ref, router_W_ref,
              router_b_ref, expert_W_ref, expert_b_ref, out_ref, coeff_ref):
    u = pl.program_id(0)

    @pl.when(u == 0)
    def _():
        xb = x_ref[...]
        scores = lax.dot_general(xb, router_W_ref[...],
                                 (((1,), (1,)), ((), ())),
                                 preferred_element_type=jnp.float32)
        scores = scores + router_b_ref[...]
        eidx = lax.broadcasted_iota(jnp.int32, scores.shape, 1)
        m0 = jnp.max(scores, axis=-1, keepdims=True)
        a0 = jnp.min(jnp.where(scores == m0, eidx, NUM_EXPERTS), axis=-1,
                     keepdims=True)
        masked = jnp.where(eidx == a0, -jnp.inf, scores)
        m1 = jnp.max(masked, axis=-1, keepdims=True)
        a1 = jnp.min(jnp.where(masked == m1, eidx, NUM_EXPERTS), axis=-1,
                     keepdims=True)
        z = jnp.exp(m1 - m0)
        w0 = 1.0 / (1.0 + z)
        w1 = z * w0
        coeff_ref[...] = (jnp.where(eidx == a0, w0, 0.0)
                          + jnp.where(eidx == a1, w1, 0.0))
        so = lax.dot_general(xb, shared_W_ref[...], (((1,), (1,)), ((), ())),
                             preferred_element_type=jnp.float32)
        out_ref[...] = so + shared_b_ref[...]

    @pl.when(u > 0)
    def _():
        xb = x_ref[...]
        call = coeff_ref[...]
        lane = lax.broadcasted_iota(jnp.int32, call.shape, 1)
        for k in range(EPAIR):
            e = (u - 1) * EPAIR + k
            eo = lax.dot_general(xb, expert_W_ref[k],
                                 (((1,), (1,)), ((), ())),
                                 preferred_element_type=jnp.float32)
            coeff = jnp.sum(jnp.where(lane == e, call, 0.0), axis=1,
                            keepdims=True)
            out_ref[...] += coeff * (eo + expert_b_ref[k])


@jax.jit
def kernel(x, shared_W, shared_b, router_W, router_b, expert_W, expert_b):
    B, S, D = x.shape
    x2 = x.reshape(S, D)

    def _w_idx(u):
        e = jnp.maximum(u - 1, 0)
        return (e, 0, 0)

    out = pl.pallas_call(
        _moe_body,
        grid=(NPAIR + 1,),
        in_specs=[
            pl.BlockSpec((S, D), lambda u: (0, 0)),
            pl.BlockSpec((D, D), lambda u: (0, 0)),
            pl.BlockSpec((1, D), lambda u: (0, 0)),
            pl.BlockSpec((NUM_EXPERTS, D), lambda u: (0, 0)),
            pl.BlockSpec((1, NUM_EXPERTS), lambda u: (0, 0)),
            pl.BlockSpec((EPAIR, D, D), _w_idx),
            pl.BlockSpec((EPAIR, 1, D), _w_idx),
        ],
        out_specs=pl.BlockSpec((S, D), lambda u: (0, 0)),
        out_shape=jax.ShapeDtypeStruct((S, D), jnp.float32),
        scratch_shapes=[pltpu.VMEM((S, NUM_EXPERTS), jnp.float32)],
    )(x2, shared_W, shared_b.reshape(1, D),
      router_W, router_b.reshape(1, NUM_EXPERTS),
      expert_W, expert_b.reshape(NUM_EXPERTS, 1, D))
    return out.reshape(B, S, D)
